# baseline (speedup 1.0000x reference)
import functools

import jax
import jax.numpy as jnp
from jax import lax
from jax.experimental import pallas as pl
from jax.experimental.pallas import tpu as pltpu
from jax.experimental.pallas import tpu_sc as plsc

_B = 2
_V = 4
_P = 2048
_S = 49152
_C = 64
_NVAR = 8

_NW = 32
_RPW = (_B * _V * _P) // _NW
_CH = 128
_NCH = _RPW // _CH
_LANES = 16
_GRP = _CH // _LANES


def _make_sc_gather():
    info = plsc.get_sparse_core_info()
    nc = info.num_cores

    mesh = plsc.VectorSubcoreMesh(core_axis_name="c", subcore_axis_name="s")

    @functools.partial(
        pl.kernel,
        mesh=mesh,
        compiler_params=pltpu.CompilerParams(needs_layout_passes=False),
        out_type=jax.ShapeDtypeStruct((_NW, _NCH, _CH, _C), jnp.float32),
        scratch_types=[
            pltpu.VMEM((_RPW,), jnp.int32),
            pltpu.VMEM((_LANES,), jnp.int32),
            pltpu.VMEM((_NCH, _CH), jnp.int32),
            pltpu.VMEM((_NCH, _CH), jnp.int32),
            pltpu.VMEM((2, _CH, 2 * _C), jnp.float32),
            pltpu.VMEM((2, _CH, _C), jnp.float32),
            pltpu.SemaphoreType.DMA,
            pltpu.SemaphoreType.DMA,
        ],
    )
    def gather_kernel(vb_hbm, patch_hbm, table_hbm, out_hbm,
                      patch_v, var_v, idx_v, half_v, pair_v, out_v,
                      sem_in, sem_out):
        wid = lax.axis_index("s") * nc + lax.axis_index("c")
        b = wid // (_NW // _B)
        p0 = (wid % (_P // _RPW)) * _RPW
        pltpu.sync_copy(patch_hbm.at[pl.ds(b * _P + p0, _RPW)], patch_v)
        pltpu.sync_copy(vb_hbm.at[wid], var_v)
        row_base = var_v[...] * _S
        per_chunk = _CH // _LANES
        for j in range(_RPW // _LANES):
            sl = patch_v[pl.ds(j * _LANES, _LANES)]
            r = row_base + sl
            c, k = j // per_chunk, (j % per_chunk) * _LANES
            idx_v[c, pl.ds(k, _LANES)] = lax.shift_right_logical(r, 1)
            half_v[c, pl.ds(k, _LANES)] = lax.shift_left(
                lax.bitwise_and(r, 1), 6)
        in_copies = [
            pltpu.make_async_copy(
                table_hbm.at[idx_v.at[c]], pair_v.at[c % 2], sem_in)
            for c in range(_NCH)
        ]
        out_copies = [
            pltpu.make_async_copy(
                out_v.at[c % 2], out_hbm.at[wid, c], sem_out)
            for c in range(_NCH)
        ]
        in_copies[0].start()
        in_copies[1].start()
        lanes = lax.iota(jnp.int32, _LANES)
        for c in range(_NCH):
            in_copies[c].wait()
            if c >= 2:
                out_copies[c - 2].wait()

            def compact_group(g, carry, c=c):
                rows = g * _LANES + lanes
                half = half_v[c, pl.ds(g * _LANES, _LANES)]
                for col in range(_C):
                    vals = plsc.load_gather(
                        pair_v.at[c % 2], [rows, half + col])
                    plsc.store_scatter(
                        out_v.at[c % 2],
                        [rows, jnp.full((_LANES,), col, jnp.int32)], vals)
                return carry

            lax.fori_loop(0, _GRP, compact_group, 0)
            out_copies[c].start()
            if c + 2 < _NCH:
                in_copies[c + 2].start()
        out_copies[_NCH - 2].wait()
        out_copies[_NCH - 1].wait()

    return gather_kernel


def kernel(var_indices, patch_idx, mg_embedding):
    table = mg_embedding.reshape(_NVAR * _S // 2, 2 * _C)
    var_flat = var_indices.reshape(-1).astype(jnp.int32)
    vb = jnp.broadcast_to(
        jnp.repeat(var_flat, _NW // (_B * _V))[:, None], (_NW, _LANES))
    patch_flat = patch_idx.reshape(-1).astype(jnp.int32)
    out = _make_sc_gather()(vb, patch_flat, table)
    return out.reshape(_B, _V, 1, _P, _C)


# speedup vs baseline: 3.5191x; 3.5191x over previous
import functools

import jax
import jax.numpy as jnp
from jax import lax
from jax.experimental import pallas as pl
from jax.experimental.pallas import tpu as pltpu
from jax.experimental.pallas import tpu_sc as plsc

_B = 2
_V = 4
_P = 2048
_S = 49152
_C = 64
_NVAR = 8

_NW = 32
_TR = _B * _V * _C
_RPW = _TR // _NW
_LANES = 16


def _make_sc_gather():
    info = plsc.get_sparse_core_info()
    nc = info.num_cores

    mesh = plsc.VectorSubcoreMesh(core_axis_name="c", subcore_axis_name="s")

    @functools.partial(
        pl.kernel,
        mesh=mesh,
        compiler_params=pltpu.CompilerParams(needs_layout_passes=False),
        out_type=jax.ShapeDtypeStruct((_TR, _P), jnp.float32),
        scratch_types=[
            pltpu.VMEM((_P,), jnp.int32),
            pltpu.VMEM((_LANES,), jnp.int32),
            pltpu.VMEM((_RPW, _LANES), jnp.int32),
            pltpu.VMEM((1, _S), jnp.float32),
            pltpu.VMEM((_P,), jnp.float32),
            pltpu.SemaphoreType.DMA,
        ],
    )
    def gather_kernel(vb_hbm, patch_hbm, table_hbm, out_hbm,
                      patch_v, var_v, rowid_v, row_v, out_v, sem):
        wid = lax.axis_index("s") * nc + lax.axis_index("c")
        bv = wid // (_NW // (_B * _V))
        b = bv // _V
        c0 = (wid % (_NW // (_B * _V))) * _RPW
        pltpu.sync_copy(patch_hbm.at[pl.ds(b * _P, _P)], patch_v)
        pltpu.sync_copy(vb_hbm.at[wid], var_v)
        base = var_v[...] * _C + c0
        for k in range(_RPW):
            rowid_v[k, :] = base + k
        for k in range(_RPW):
            pltpu.async_copy(
                table_hbm.at[rowid_v.at[k, pl.ds(0, 1)]], row_v, sem).wait()
            row = row_v.at[0]

            def gather_group(g, carry):
                idx = patch_v[pl.ds(g * _LANES, _LANES)]
                out_v[pl.ds(g * _LANES, _LANES)] = plsc.load_gather(row, [idx])
                return carry

            lax.fori_loop(0, _P // _LANES, gather_group, 0)
            pltpu.sync_copy(out_v, out_hbm.at[wid * _RPW + k])

    return gather_kernel


def kernel(var_indices, patch_idx, mg_embedding):
    table = jnp.transpose(mg_embedding, (0, 2, 1)).reshape(_NVAR * _C, _S)
    var_flat = var_indices.reshape(-1).astype(jnp.int32)
    vb = jnp.broadcast_to(
        jnp.repeat(var_flat, _NW // (_B * _V))[:, None], (_NW, _LANES))
    patch_flat = patch_idx.reshape(-1).astype(jnp.int32)
    out = _make_sc_gather()(vb, patch_flat, table)
    return jnp.transpose(
        out.reshape(_B, _V, _C, _P), (0, 1, 3, 2))[:, :, None, :, :]


# speedup vs baseline: 3.7749x; 1.0727x over previous
import functools

import jax
import jax.numpy as jnp
from jax import lax
from jax.experimental import pallas as pl
from jax.experimental.pallas import tpu as pltpu
from jax.experimental.pallas import tpu_sc as plsc

_B = 2
_V = 4
_P = 2048
_S = 49152
_C = 64
_NVAR = 8

_NW = 32
_NSLOT = _B * _V
_LANES = 16
_UNROLL = 8
_GRPS = _P // _LANES


def _make_sc_gather():
    info = plsc.get_sparse_core_info()
    nc = info.num_cores

    mesh = plsc.VectorSubcoreMesh(core_axis_name="c", subcore_axis_name="s")

    @functools.partial(
        pl.kernel,
        mesh=mesh,
        compiler_params=pltpu.CompilerParams(needs_layout_passes=False),
        out_type=jax.ShapeDtypeStruct((_NSLOT * _C, _P), jnp.float32),
        scratch_types=[
            pltpu.VMEM((_B, _P), jnp.int32),
            pltpu.VMEM((_LANES,), jnp.int32),
            pltpu.VMEM((_NSLOT, _LANES), jnp.int32),
            pltpu.VMEM((_NSLOT, _LANES), jnp.int32),
            pltpu.VMEM((_NSLOT, _NSLOT, _LANES), jnp.int32),
            pltpu.VMEM((_S,), jnp.float32),
            pltpu.VMEM((_P,), jnp.float32),
        ],
    )
    def gather_kernel(dv_hbm, uv_hbm, ns_hbm, sl_hbm, patch_hbm, table_hbm,
                      out_hbm, patch_v, dv_v, uv_v, ns_v, sl_v, row_v, out_v):
        wid = lax.axis_index("s") * nc + lax.axis_index("c")
        pltpu.sync_copy(patch_hbm, patch_v)
        pltpu.sync_copy(dv_hbm, dv_v)
        pltpu.sync_copy(uv_hbm, uv_v)
        pltpu.sync_copy(ns_hbm, ns_v)
        pltpu.sync_copy(sl_hbm, sl_v)
        d_cnt = lax.reduce_max(dv_v[...], axes=(0,))

        def row_body(k, carry):
            r = wid + _NW * k
            d = r // _C
            c = lax.rem(r, _C)
            uvar = lax.reduce_max(uv_v[d], axes=(0,))
            pltpu.sync_copy(table_hbm.at[uvar * _C + c], row_v)
            n_d = lax.reduce_max(ns_v[d], axes=(0,))

            def slot_body(j, carry2):
                slot = lax.reduce_max(sl_v[d, j], axes=(0,))
                b_j = slot // _V

                def g_body(g, carry3):
                    for u in range(_UNROLL):
                        o = (g * _UNROLL + u) * _LANES
                        idx = patch_v[b_j, pl.ds(o, _LANES)]
                        out_v[pl.ds(o, _LANES)] = plsc.load_gather(
                            row_v, [idx])
                    return carry3

                lax.fori_loop(0, _GRPS // _UNROLL, g_body, 0)
                pltpu.sync_copy(out_v, out_hbm.at[slot * _C + c])
                return carry2

            lax.fori_loop(0, n_d, slot_body, 0)
            return carry

        lax.fori_loop(0, (d_cnt * _C) // _NW, row_body, 0)

    return gather_kernel


def kernel(var_indices, patch_idx, mg_embedding):
    table = jnp.transpose(mg_embedding, (0, 2, 1)).reshape(_NVAR * _C, _S)
    var_flat = var_indices.reshape(-1).astype(jnp.int32)
    patch2 = patch_idx.reshape(_B, _P).astype(jnp.int32)

    slots = jnp.arange(_NSLOT, dtype=jnp.int32)
    eq = var_flat[None, :] == var_flat[:, None]
    first = jnp.argmax(eq, axis=1).astype(jnp.int32)
    is_leader = first == slots
    d_cnt = is_leader.sum(dtype=jnp.int32)
    pos = jnp.cumsum(is_leader, dtype=jnp.int32) - 1
    uvars = jnp.zeros(_NSLOT, jnp.int32).at[
        jnp.where(is_leader, pos, _NSLOT)].set(var_flat, mode="drop")
    g_of_slot = pos[first]
    match = g_of_slot[None, :] == slots[:, None]
    nslots = match.sum(axis=1, dtype=jnp.int32)
    slotlist = jnp.argsort(~match, axis=1, stable=True).astype(jnp.int32)

    def bc(a):
        return jnp.broadcast_to(a[..., None], a.shape + (_LANES,))

    out = _make_sc_gather()(
        bc(d_cnt[None])[0], bc(uvars), bc(nslots), bc(slotlist),
        patch2, table)
    return jnp.transpose(
        out.reshape(_B, _V, _C, _P), (0, 1, 3, 2))[:, :, None, :, :]


# speedup vs baseline: 4.2961x; 1.1381x over previous
import functools

import jax
import jax.numpy as jnp
from jax import lax
from jax.experimental import pallas as pl
from jax.experimental.pallas import tpu as pltpu
from jax.experimental.pallas import tpu_sc as plsc

_B = 2
_V = 4
_P = 2048
_S = 49152
_C = 64
_NVAR = 8

_NW = 32
_NSLOT = _B * _V
_LANES = 16
_UNROLL = 16
_GRPS = _P // _LANES
_PAD = 127


def _make_sc_gather():
    info = plsc.get_sparse_core_info()
    nc = info.num_cores

    mesh = plsc.VectorSubcoreMesh(core_axis_name="c", subcore_axis_name="s")

    @functools.partial(
        pl.kernel,
        mesh=mesh,
        compiler_params=pltpu.CompilerParams(needs_layout_passes=False),
        out_type=jax.ShapeDtypeStruct((_NSLOT * _C, _P), jnp.float32),
        scratch_types=[
            pltpu.VMEM((_B, _P), jnp.int32),
            pltpu.VMEM((_LANES,), jnp.int32),
            pltpu.VMEM((2, _S), jnp.float32),
            pltpu.VMEM((2, _P), jnp.float32),
            pltpu.SemaphoreType.DMA,
            pltpu.SemaphoreType.DMA,
        ],
    )
    def gather_kernel(varp_hbm, patch_hbm, table_hbm, out_hbm,
                      patch_v, varp_v, row_v, out_v, sem_in, sem_out):
        wid = lax.axis_index("s") * nc + lax.axis_index("c")
        pltpu.sync_copy(patch_hbm, patch_v)
        pltpu.sync_copy(varp_hbm, varp_v)
        lanes = lax.iota(jnp.int32, _LANES)
        var_vec = varp_v[...]

        first = lanes
        for t in range(_NSLOT - 1, -1, -1):
            vt = lax.reduce_max(
                jnp.where(lanes == t, var_vec, -1), axes=(0,))
            first = jnp.where(var_vec == vt, t, first)
        is_leader = jnp.logical_and(first == lanes, lanes < _NSLOT)
        pos = plsc.cumsum(jnp.where(is_leader, 1, 0).astype(jnp.int32)) - 1
        d_cnt = lax.reduce_max(pos, axes=(0,)) + 1
        nrows = (d_cnt * _C) // _NW

        def rowid(k):
            r = wid + _NW * k
            d = r // _C
            c = lax.rem(r, _C)
            sel = jnp.logical_and(pos == d, is_leader)
            uvar = lax.reduce_max(
                jnp.where(sel, var_vec, -1), axes=(0,))
            return d, c, uvar

        def start_row(k, buf):
            _, c, uvar = rowid(k)
            pltpu.make_async_copy(
                table_hbm.at[pl.ds(uvar * _C + c, 1)],
                row_v.at[pl.ds(buf, 1)], sem_in).start()

        start_row(0, 0)
        start_row(1, 1)

        def row_body(k, m):
            d, c, uvar = rowid(k)
            buf = lax.rem(k, 2)
            bufv = jnp.full((_LANES,), buf, jnp.int32)
            pltpu.make_async_copy(
                table_hbm.at[pl.ds(0, 1)], row_v.at[pl.ds(buf, 1)],
                sem_in).wait()
            match = var_vec == uvar
            match_i = jnp.where(match, 1, 0).astype(jnp.int32)
            n_d = jnp.sum(match_i)
            rank = plsc.cumsum(match_i) - 1

            def slot_body(j, m2):
                slot = lax.reduce_max(
                    jnp.where(jnp.logical_and(match, rank == j), lanes, -1),
                    axes=(0,))
                b_j = slot // _V
                par = lax.rem(m2, 2)
                parv = jnp.full((_LANES,), par, jnp.int32)

                @pl.when(m2 >= 2)
                def _():
                    pltpu.make_async_copy(
                        out_v.at[pl.ds(par, 1)], out_hbm.at[pl.ds(0, 1)],
                        sem_out).wait()

                def g_body(g, carry3):
                    for u in range(_UNROLL):
                        o = (g * _UNROLL + u) * _LANES
                        idx = patch_v[b_j, pl.ds(o, _LANES)]
                        out_v[par, pl.ds(o, _LANES)] = plsc.load_gather(
                            row_v, [bufv, idx])
                    return carry3

                lax.fori_loop(0, _GRPS // _UNROLL, g_body, 0)
                pltpu.make_async_copy(
                    out_v.at[pl.ds(par, 1)],
                    out_hbm.at[pl.ds(slot * _C + c, 1)], sem_out).start()
                return m2 + 1

            m = lax.fori_loop(0, n_d, slot_body, m)

            @pl.when(k + 2 < nrows)
            def _():
                start_row(k + 2, buf)

            return m

        m = lax.fori_loop(0, nrows, row_body, 0)
        pltpu.make_async_copy(
            out_v.at[pl.ds(0, 1)], out_hbm.at[pl.ds(0, 1)], sem_out).wait()
        pltpu.make_async_copy(
            out_v.at[pl.ds(0, 1)], out_hbm.at[pl.ds(0, 1)], sem_out).wait()

    return gather_kernel


def kernel(var_indices, patch_idx, mg_embedding):
    table = jnp.transpose(mg_embedding, (0, 2, 1)).reshape(_NVAR * _C, _S)
    var_flat = var_indices.reshape(-1).astype(jnp.int32)
    varp = jnp.concatenate(
        [var_flat, jnp.full((_LANES - _NSLOT,), _PAD, jnp.int32)])
    patch2 = patch_idx.reshape(_B, _P).astype(jnp.int32)
    out = _make_sc_gather()(varp, patch2, table)
    return jnp.transpose(
        out.reshape(_B, _V, _C, _P), (0, 1, 3, 2))[:, :, None, :, :]


# speedup vs baseline: 6.1005x; 1.4200x over previous
import functools

import jax
import jax.numpy as jnp
from jax import lax
from jax.experimental import pallas as pl
from jax.experimental.pallas import tpu as pltpu
from jax.experimental.pallas import tpu_sc as plsc

_B = 2
_V = 4
_P = 2048
_S = 49152
_C = 64
_NVAR = 8

_NW = 32
_NSLOT = _B * _V
_LANES = 16
_UNROLL = 16
_GRPS = _P // _LANES
_PAD = 127


def _make_sc_gather():
    info = plsc.get_sparse_core_info()
    nc = info.num_cores

    mesh = plsc.VectorSubcoreMesh(core_axis_name="c", subcore_axis_name="s")

    @functools.partial(
        pl.kernel,
        mesh=mesh,
        compiler_params=pltpu.CompilerParams(needs_layout_passes=False),
        out_type=jax.ShapeDtypeStruct((_NSLOT * _C, _P), jnp.float32),
        scratch_types=[
            pltpu.VMEM((_B, _P), jnp.int32),
            pltpu.VMEM((_LANES,), jnp.int32),
            pltpu.VMEM((2, _S), jnp.float32),
            pltpu.VMEM((2, _P), jnp.float32),
            pltpu.SemaphoreType.DMA,
            pltpu.SemaphoreType.DMA,
        ],
    )
    def gather_kernel(varp_hbm, patch_hbm, table_hbm, out_hbm,
                      patch_v, varp_v, row_v, out_v, sem_in, sem_out):
        wid = lax.axis_index("s") * nc + lax.axis_index("c")
        pltpu.sync_copy(patch_hbm, patch_v)
        pltpu.sync_copy(varp_hbm, varp_v)
        lanes = lax.iota(jnp.int32, _LANES)
        var_vec = varp_v[...]

        first = lanes
        for t in range(_NSLOT - 1, -1, -1):
            vt = lax.reduce_max(
                jnp.where(lanes == t, var_vec, -1), axes=(0,))
            first = jnp.where(var_vec == vt, t, first)
        is_leader = jnp.logical_and(first == lanes, lanes < _NSLOT)
        pos = plsc.cumsum(jnp.where(is_leader, 1, 0).astype(jnp.int32)) - 1
        d_cnt = lax.reduce_max(pos, axes=(0,)) + 1
        nrows = (d_cnt * _C) // _NW

        def rowid(k):
            r = wid + _NW * k
            d = r // _C
            c = lax.rem(r, _C)
            sel = jnp.logical_and(pos == d, is_leader)
            uvar = lax.reduce_max(
                jnp.where(sel, var_vec, -1), axes=(0,))
            return d, c, uvar

        def start_row(k, buf):
            _, c, uvar = rowid(k)
            pltpu.make_async_copy(
                table_hbm.at[pl.ds(uvar * _C + c, 1)],
                row_v.at[pl.ds(buf, 1)], sem_in).start()

        start_row(0, 0)
        start_row(1, 1)

        def row_body(k, m):
            d, c, uvar = rowid(k)
            buf = lax.rem(k, 2)
            bufv = jnp.full((_LANES,), buf, jnp.int32)
            pltpu.make_async_copy(
                table_hbm.at[pl.ds(0, 1)], row_v.at[pl.ds(buf, 1)],
                sem_in).wait()
            match = var_vec == uvar
            match_i = jnp.where(match, 1, 0).astype(jnp.int32)
            n_d = jnp.sum(match_i)
            rank = plsc.cumsum(match_i) - 1

            def slot_body(j, m2):
                slot = lax.reduce_max(
                    jnp.where(jnp.logical_and(match, rank == j), lanes, -1),
                    axes=(0,))
                b_j = slot // _V
                par = lax.rem(m2, 2)
                parv = jnp.full((_LANES,), par, jnp.int32)

                @pl.when(m2 >= 2)
                def _():
                    pltpu.make_async_copy(
                        out_v.at[pl.ds(par, 1)], out_hbm.at[pl.ds(0, 1)],
                        sem_out).wait()

                @plsc.parallel_loop(0, _GRPS, step=1, unroll=_UNROLL)
                def _gather(g):
                    o = g * _LANES
                    idx = patch_v[b_j, pl.ds(o, _LANES)]
                    out_v[par, pl.ds(o, _LANES)] = plsc.load_gather(
                        row_v, [bufv, idx])
                pltpu.make_async_copy(
                    out_v.at[pl.ds(par, 1)],
                    out_hbm.at[pl.ds(slot * _C + c, 1)], sem_out).start()
                return m2 + 1

            m = lax.fori_loop(0, n_d, slot_body, m)

            @pl.when(k + 2 < nrows)
            def _():
                start_row(k + 2, buf)

            return m

        m = lax.fori_loop(0, nrows, row_body, 0)
        pltpu.make_async_copy(
            out_v.at[pl.ds(0, 1)], out_hbm.at[pl.ds(0, 1)], sem_out).wait()
        pltpu.make_async_copy(
            out_v.at[pl.ds(0, 1)], out_hbm.at[pl.ds(0, 1)], sem_out).wait()

    return gather_kernel


def kernel(var_indices, patch_idx, mg_embedding):
    table = jnp.transpose(mg_embedding, (0, 2, 1)).reshape(_NVAR * _C, _S)
    var_flat = var_indices.reshape(-1).astype(jnp.int32)
    varp = jnp.concatenate(
        [var_flat, jnp.full((_LANES - _NSLOT,), _PAD, jnp.int32)])
    patch2 = patch_idx.reshape(_B, _P).astype(jnp.int32)
    out = _make_sc_gather()(varp, patch2, table)
    return jnp.transpose(
        out.reshape(_B, _V, _C, _P), (0, 1, 3, 2))[:, :, None, :, :]
